# Initial kernel scaffold; baseline (speedup 1.0000x reference)
import jax
import jax.numpy as jnp
from jax.experimental import pallas as pl


def kernel(node_h, edge_h, src, dst, emb_h_w, emb_h_b, emb_e_w, emb_e_b, w4, b4, wc, bc, bn_h_g, bn_h_b, bn_e_g, bn_e_b, mlp_w0, mlp_b0, mlp_w1, mlp_b1, mlp_w2, mlp_b2):
    raise NotImplementedError("write your pallas kernel here")


# speedup vs baseline: 1.0100x; 1.0100x over previous
import functools

import jax
import jax.numpy as jnp
from jax.experimental import pallas as pl
from jax.experimental.pallas import tpu as pltpu

HP = 128


def _net_kernel(node_ref, edge_ref, srcc_ref, dstc_ref, dstr_ref,
                ehw_ref, ehb_ref, eew_ref, eeb_ref,
                w4_ref, b4_ref, wc_ref, bc_ref,
                gh_ref, bth_ref, ge_ref, bte_ref,
                m0w_ref, m0b_ref, m1w_ref, m1b_ref, m2w_ref, m2b_ref,
                o_ref,
                h_scr, e_scr, soh_scr, doh_scr, doht_scr,
                *, hp, n_nodes, n_edges):
    l = pl.program_id(0)
    dot = lambda a, b: jnp.dot(a, b, preferred_element_type=jnp.float32)
    bf = jnp.bfloat16

    @pl.when(l == 0)
    def _init():
        col_en = jax.lax.broadcasted_iota(jnp.int32, (n_edges, n_nodes), 1)
        soh_scr[...] = (col_en == srcc_ref[...]).astype(bf)
        doh_scr[...] = (col_en == dstc_ref[...]).astype(bf)
        row_ne = jax.lax.broadcasted_iota(jnp.int32, (n_nodes, n_edges), 0)
        doht_scr[...] = (row_ne == dstr_ref[...]).astype(bf)
        h_scr[...] = dot(node_ref[...], ehw_ref[...]) + ehb_ref[...]
        e_scr[...] = edge_ref[...] * eew_ref[...] + eeb_ref[...]

    h = h_scr[...]
    e = e_scr[...]

    proj = dot(h, w4_ref[...]) + b4_ref[...]
    Ce = dot(e, wc_ref[...]) + bc_ref[...]

    DB_src = dot(soh_scr[...], proj[:, 0:2 * hp].astype(bf))
    Eh_dst = dot(doh_scr[...], proj[:, 3 * hp:4 * hp].astype(bf))

    e_new = DB_src[:, 0:hp] + Eh_dst + Ce
    sigma = jax.nn.sigmoid(e_new)

    msg = jnp.concatenate([sigma * DB_src[:, hp:2 * hp], sigma], axis=1)
    agg = dot(doht_scr[...], msg.astype(bf))
    h_new = proj[:, 2 * hp:3 * hp] + agg[:, 0:hp] / (agg[:, hp:2 * hp] + 1e-6)

    def bn(x, gamma, beta):
        mu = jnp.mean(x, axis=0, keepdims=True)
        xc = x - mu
        var = jnp.mean(xc * xc, axis=0, keepdims=True)
        return xc * jax.lax.rsqrt(var + 1e-5) * gamma + beta

    h_new = jnp.maximum(bn(h_new, gh_ref[...], bth_ref[...]), 0.0)
    e_new = jnp.maximum(bn(e_new, ge_ref[...], bte_ref[...]), 0.0)

    h_res = h + h_new
    h_scr[...] = h_res
    e_scr[...] = e + e_new

    @pl.when(l == pl.num_programs(0) - 1)
    def _readout():
        y = jnp.maximum(dot(h_res, m0w_ref[...]) + m0b_ref[...], 0.0)
        y = jnp.maximum(dot(y, m1w_ref[...]) + m1b_ref[...], 0.0)
        y = dot(y, m2w_ref[...]) + m2b_ref[...]
        n = jnp.sqrt(jnp.sum(y * y, axis=1, keepdims=True))
        o_ref[...] = y / jnp.maximum(n, 1e-12)


@jax.jit
def _forward(node_h, edge_h, src, dst, params):
    N = node_h.shape[0]
    E = edge_h.shape[0]
    L = params["w4"].shape[0]
    hp = params["w4"].shape[1]

    node_p = jnp.pad(node_h, ((0, 0), (0, hp - node_h.shape[1])))
    srcc = src.astype(jnp.int32).reshape(E, 1)
    dstc = dst.astype(jnp.int32).reshape(E, 1)
    dstr = dst.astype(jnp.int32).reshape(1, E)

    const2 = lambda shape: pl.BlockSpec(shape, lambda l: (0, 0))
    per_layer = lambda s1, s2: pl.BlockSpec((None, s1, s2),
                                            lambda l: (l, 0, 0))

    grid_spec = pltpu.PrefetchScalarGridSpec(
        num_scalar_prefetch=0,
        grid=(L,),
        in_specs=[
            const2((N, hp)),
            const2((E, 1)),
            const2((E, 1)),
            const2((E, 1)),
            const2((1, E)),
            const2((hp, hp)),
            const2((1, hp)),
            const2((1, hp)),
            const2((1, hp)),
            per_layer(hp, 4 * hp),
            per_layer(1, 4 * hp),
            per_layer(hp, hp),
            per_layer(1, hp),
            per_layer(1, hp),
            per_layer(1, hp),
            per_layer(1, hp),
            per_layer(1, hp),
            const2((hp, hp)),
            const2((1, hp)),
            const2((hp, hp)),
            const2((1, hp)),
            const2((hp, hp)),
            const2((1, hp)),
        ],
        out_specs=pl.BlockSpec((N, hp), lambda l: (0, 0)),
        scratch_shapes=[
            pltpu.VMEM((N, hp), jnp.float32),
            pltpu.VMEM((E, hp), jnp.float32),
            pltpu.VMEM((E, N), jnp.bfloat16),
            pltpu.VMEM((E, N), jnp.bfloat16),
            pltpu.VMEM((N, E), jnp.bfloat16),
        ],
    )
    y = pl.pallas_call(
        functools.partial(_net_kernel, hp=hp, n_nodes=N, n_edges=E),
        out_shape=jax.ShapeDtypeStruct((N, hp), jnp.float32),
        grid_spec=grid_spec,
        compiler_params=pltpu.CompilerParams(
            dimension_semantics=("arbitrary",)),
    )(node_p, edge_h, srcc, dstc, dstr,
      params["emb_h_w"], params["emb_h_b"], params["emb_e_w"],
      params["emb_e_b"],
      params["w4"], params["b4"], params["wc"], params["bc"],
      params["bn_h_g"], params["bn_h_b"], params["bn_e_g"], params["bn_e_b"],
      params["mlp_w0"], params["mlp_b0"], params["mlp_w1"], params["mlp_b1"],
      params["mlp_w2"], params["mlp_b2"])
    return y[:, :4]


def kernel(node_h, edge_h, src, dst,
           emb_h_w, emb_h_b, emb_e_w, emb_e_b,
           w4, b4, wc, bc,
           bn_h_g, bn_h_b, bn_e_g, bn_e_b,
           mlp_w0, mlp_b0, mlp_w1, mlp_b1, mlp_w2, mlp_b2):
    params = {
        "emb_h_w": emb_h_w, "emb_h_b": emb_h_b,
        "emb_e_w": emb_e_w, "emb_e_b": emb_e_b,
        "w4": w4, "b4": b4, "wc": wc, "bc": bc,
        "bn_h_g": bn_h_g, "bn_h_b": bn_h_b,
        "bn_e_g": bn_e_g, "bn_e_b": bn_e_b,
        "mlp_w0": mlp_w0, "mlp_b0": mlp_b0,
        "mlp_w1": mlp_w1, "mlp_b1": mlp_b1,
        "mlp_w2": mlp_w2, "mlp_b2": mlp_b2,
    }
    return _forward(node_h, edge_h, src, dst, params)
